# Initial kernel scaffold; baseline (speedup 1.0000x reference)
import jax
import jax.numpy as jnp
from jax.experimental import pallas as pl


def kernel(update, state):
    raise NotImplementedError("write your pallas kernel here")


# speedup vs baseline: 257.4600x; 257.4600x over previous
import functools

import jax
import jax.numpy as jnp
from jax import lax
from jax.experimental import pallas as pl
from jax.experimental.pallas import tpu as pltpu
from jax.experimental.pallas import tpu_sc as plsc

U = 512
B = 128
D = 8
C = 2
K = 8192
W = D * U
OUT = B * U
ST = (D - 1) * U
FULL = OUT + ST
NK = FULL // U
NCH = C * NK
L = 16
NG = U // L

_NC = 2
_NS = 16
_NW = _NC * _NS
_CPW = -(-NCH // _NW)


def _zero_row(ref2d, d):
    z = jnp.zeros((L,), jnp.float32)
    for i in range(NG):
        ref2d[d, pl.ds(i * L, L)] = z


def _zero_vec(ref1d):
    z = jnp.zeros((L,), jnp.float32)
    for i in range(NG):
        ref1d[pl.ds(i * L, L)] = z


def _sc_body(upd_hbm, st_hbm, out0_hbm, out1_hbm, buf, sbuf, obuf, sem):
    wid = lax.axis_index("s") * _NC + lax.axis_index("c")

    def chunk_body(j, carry):
        ch = j * _NW + wid

        @pl.when(ch < NCH)
        def _():
            c = ch // NK
            k = ch % NK

            for d in range(D):
                b = jnp.clip(k - d, 0, B - 1)
                off = (b * C + c) * K + W + d * U
                pltpu.make_async_copy(
                    upd_hbm.at[pl.ds(off, U)], buf.at[d], sem
                ).start()
            st_off = c * ST + jnp.minimum(k, D - 2) * U
            pltpu.make_async_copy(
                st_hbm.at[pl.ds(st_off, U)], sbuf, sem
            ).start()

            for d in range(D):
                pltpu.make_async_copy(
                    upd_hbm.at[pl.ds(0, U)], buf.at[d], sem
                ).wait()
            pltpu.make_async_copy(st_hbm.at[pl.ds(0, U)], sbuf, sem).wait()

            for d in range(D):
                @pl.when((k - d < 0) | (k - d > B - 1))
                def _(d=d):
                    _zero_row(buf, d)

            @pl.when(k >= D - 1)
            def _():
                _zero_vec(sbuf)

            for i in range(NG):
                s = buf[0, pl.ds(i * L, L)]
                for d in range(1, D):
                    s = s + buf[d, pl.ds(i * L, L)]
                obuf[pl.ds(i * L, L)] = s * jnp.float32(1.0 / D) + sbuf[
                    pl.ds(i * L, L)
                ]

            @pl.when(k < B)
            def _():
                pltpu.sync_copy(obuf, out0_hbm.at[pl.ds(c * OUT + k * U, U)])

            @pl.when(k >= B)
            def _():
                pltpu.sync_copy(
                    obuf, out1_hbm.at[pl.ds(c * ST + (k - B) * U, U)]
                )

        return carry

    lax.fori_loop(0, _CPW, chunk_body, 0)


@jax.jit
def _sc_call(upd_flat, st_flat):
    mesh = plsc.VectorSubcoreMesh(core_axis_name="c", subcore_axis_name="s")
    return pl.kernel(
        _sc_body,
        out_type=(
            jax.ShapeDtypeStruct((C * OUT,), jnp.float32),
            jax.ShapeDtypeStruct((C * ST,), jnp.float32),
        ),
        mesh=mesh,
        scratch_types=[
            pltpu.VMEM((D, U), jnp.float32),
            pltpu.VMEM((U,), jnp.float32),
            pltpu.VMEM((U,), jnp.float32),
            pltpu.SemaphoreType.DMA,
        ],
    )(upd_flat, st_flat)


def kernel(update, state):
    out0, out1 = _sc_call(update.reshape(-1), state.reshape(-1))
    return out0.reshape(C, OUT), out1.reshape(C, ST)


# speedup vs baseline: 371.1710x; 1.4417x over previous
import jax
import jax.numpy as jnp
from jax import lax
from jax.experimental import pallas as pl
from jax.experimental.pallas import tpu as pltpu
from jax.experimental.pallas import tpu_sc as plsc

U = 512
B = 128
D = 8
C = 2
K = 8192
W = D * U
OUT = B * U
ST = (D - 1) * U
FULL = OUT + ST
NK = FULL // U
NCH = C * NK
L = 16
NG = U // L

_NC = 2
_NS = 16
_NW = _NC * _NS
_CPW = -(-NCH // _NW)


def _zero_row(ref2d, d):
    z = jnp.zeros((L,), jnp.float32)
    for i in range(NG):
        ref2d[d, pl.ds(i * L, L)] = z


def _zero_vec(ref1d):
    z = jnp.zeros((L,), jnp.float32)
    for i in range(NG):
        ref1d[pl.ds(i * L, L)] = z


def _sc_body(upd_hbm, st_hbm, out0_hbm, out1_hbm, buf, sbuf, obuf, sem):
    wid = lax.axis_index("s") * _NC + lax.axis_index("c")

    def chunk_body(j, carry):
        ch = j * _NW + wid

        @pl.when(ch < NCH)
        def _():
            c = ch // NK
            k = ch % NK

            for d in range(D):
                b = jnp.clip(k - d, 0, B - 1)
                pltpu.make_async_copy(
                    upd_hbm.at[b, c, pl.ds(W + d * U, U)], buf.at[d], sem
                ).start()
            sk = jnp.minimum(k, D - 2)
            pltpu.make_async_copy(
                st_hbm.at[c, pl.ds(sk * U, U)], sbuf, sem
            ).start()

            for d in range(D):
                pltpu.make_async_copy(
                    upd_hbm.at[0, 0, pl.ds(0, U)], buf.at[d], sem
                ).wait()
            pltpu.make_async_copy(st_hbm.at[0, pl.ds(0, U)], sbuf, sem).wait()

            for d in range(D):
                @pl.when((k - d < 0) | (k - d > B - 1))
                def _(d=d):
                    _zero_row(buf, d)

            @pl.when(k >= D - 1)
            def _():
                _zero_vec(sbuf)

            for i in range(NG):
                s = buf[0, pl.ds(i * L, L)]
                for d in range(1, D):
                    s = s + buf[d, pl.ds(i * L, L)]
                obuf[pl.ds(i * L, L)] = s * jnp.float32(1.0 / D) + sbuf[
                    pl.ds(i * L, L)
                ]

            @pl.when(k < B)
            def _():
                pltpu.sync_copy(obuf, out0_hbm.at[c, pl.ds(k * U, U)])

            @pl.when(k >= B)
            def _():
                pltpu.sync_copy(obuf, out1_hbm.at[c, pl.ds((k - B) * U, U)])

        return carry

    lax.fori_loop(0, _CPW, chunk_body, 0)


@jax.jit
def kernel(update, state):
    mesh = plsc.VectorSubcoreMesh(core_axis_name="c", subcore_axis_name="s")
    return pl.kernel(
        _sc_body,
        out_type=(
            jax.ShapeDtypeStruct((C, OUT), jnp.float32),
            jax.ShapeDtypeStruct((C, ST), jnp.float32),
        ),
        mesh=mesh,
        scratch_types=[
            pltpu.VMEM((D, U), jnp.float32),
            pltpu.VMEM((U,), jnp.float32),
            pltpu.VMEM((U,), jnp.float32),
            pltpu.SemaphoreType.DMA,
        ],
    )(update, state)
